# initial kernel scaffold (unmeasured)
import functools

import jax
import jax.numpy as jnp
from jax import lax
from jax.experimental import pallas as pl
from jax.experimental.pallas import tpu as pltpu

N_DEV = 32
M_PER = 128
K = 4096
N_COLS = 8192
N_PER = 256
CHUNK = 512
N_CHUNKS = N_COLS // CHUNK
TILES_PER_CHUNK = CHUNK // N_PER


def kernel(x, w_mat, scale_x, scale_w):
    def body(x_ref, w_ref, sx_ref, sw_ref, out_ref,
             wbuf, y_ref, copy_sems, send_sems, recv_sems):
        my = lax.axis_index("i")
        scale = sx_ref[0] * sw_ref[0]
        x_bf = x_ref[...].astype(jnp.bfloat16)

        def start_copy(c, slot):
            cp = pltpu.make_async_copy(
                w_ref.at[:, pl.ds(c * CHUNK, CHUNK)],
                wbuf.at[slot],
                copy_sems.at[slot],
            )
            cp.start()
            return cp

        copies = [start_copy(0, 0), start_copy(1, 1)]
        sends = []
        for c in range(N_CHUNKS):
            slot = c % 2
            copies[slot].wait()
            acc = jnp.dot(
                x_bf, wbuf[slot].astype(jnp.bfloat16),
                preferred_element_type=jnp.float32,
            )
            if c + 2 < N_CHUNKS:
                copies[slot] = start_copy(c + 2, slot)
            yv = acc * scale
            yv = yv / (1.0 + jnp.exp(-jnp.clip(yv, -60.0, 60.0)))
            y_ref[:, pl.ds(c * CHUNK, CHUNK)] = yv

        out_ref[pl.ds(my * M_PER, M_PER), :] = (
            y_ref[:, pl.ds(my * N_PER, N_PER)]
        )

        for h in range(1, N_DEV):
            j = lax.rem(my + h, N_DEV)
            rdma = pltpu.make_async_remote_copy(
                src_ref=y_ref.at[:, pl.ds(j * N_PER, N_PER)],
                dst_ref=out_ref.at[pl.ds(my * M_PER, M_PER), :],
                send_sem=send_sems.at[h - 1],
                recv_sem=recv_sems.at[my],
                device_id=(j,),
                device_id_type=pl.DeviceIdType.MESH,
            )
            rdma.start()
            sends.append(rdma)

        for h in range(1, N_DEV):
            p = lax.rem(my + N_DEV - h, N_DEV)
            recv = pltpu.make_async_remote_copy(
                src_ref=y_ref.at[:, pl.ds(p * N_PER, N_PER)],
                dst_ref=out_ref.at[pl.ds(p * M_PER, M_PER), :],
                send_sem=send_sems.at[h - 1],
                recv_sem=recv_sems.at[p],
                device_id=(p,),
                device_id_type=pl.DeviceIdType.MESH,
            )
            recv.wait_recv()

        for rdma in sends:
            rdma.wait_send()

    out_shape = jax.ShapeDtypeStruct((N_DEV * M_PER, N_PER), jnp.float32)
    return pl.pallas_call(
        body,
        out_shape=out_shape,
        in_specs=[
            pl.BlockSpec(memory_space=pltpu.VMEM),
            pl.BlockSpec(memory_space=pltpu.ANY),
            pl.BlockSpec(memory_space=pltpu.SMEM),
            pl.BlockSpec(memory_space=pltpu.SMEM),
        ],
        out_specs=pl.BlockSpec(memory_space=pltpu.VMEM),
        scratch_shapes=[
            pltpu.VMEM((2, K, CHUNK), jnp.float32),
            pltpu.VMEM((M_PER, N_COLS), jnp.float32),
            pltpu.SemaphoreType.DMA((2,)),
            pltpu.SemaphoreType.DMA((N_DEV - 1,)),
            pltpu.SemaphoreType.DMA((N_DEV,)),
        ],
        compiler_params=pltpu.CompilerParams(collective_id=0),
    )(x, w_mat, scale_x, scale_w)


# baseline (device time: 108692 ns/iter reference)
import functools

import jax
import jax.numpy as jnp
from jax import lax
from jax.experimental import pallas as pl
from jax.experimental.pallas import tpu as pltpu

N_DEV = 32
M_PER = 128
K = 4096
N_COLS = 8192
N_PER = 256
CHUNK = 512
N_CHUNKS = N_COLS // CHUNK
TILES_PER_CHUNK = CHUNK // N_PER


def kernel(x, w_mat, scale_x, scale_w):
    def body(x_ref, w_ref, sx_ref, sw_ref, out_ref,
             wbuf, y_ref, copy_sems, send_sems, recv_sems):
        my = lax.axis_index("i")
        scale = sx_ref[0] * sw_ref[0]
        x_bf = x_ref[...].astype(jnp.bfloat16)

        def start_copy(c, slot):
            cp = pltpu.make_async_copy(
                w_ref.at[:, pl.ds(c * CHUNK, CHUNK)],
                wbuf.at[slot],
                copy_sems.at[slot],
            )
            cp.start()
            return cp

        copies = [start_copy(0, 0), start_copy(1, 1)]
        sends = []
        for c in range(N_CHUNKS):
            slot = c % 2
            copies[slot].wait()
            acc = jnp.dot(
                x_bf, wbuf[slot].astype(jnp.bfloat16),
                preferred_element_type=jnp.float32,
            )
            if c + 2 < N_CHUNKS:
                copies[slot] = start_copy(c + 2, slot)
            yv = acc * scale
            yv = yv / (1.0 + jnp.exp(-jnp.clip(yv, -60.0, 60.0)))
            y_ref[:, pl.ds(c * CHUNK, CHUNK)] = yv

        out_ref[pl.ds(my * M_PER, M_PER), :] = (
            y_ref[:, pl.ds(my * N_PER, N_PER)]
        )

        for h in range(1, N_DEV):
            j = lax.rem(my + h, N_DEV)
            rdma = pltpu.make_async_remote_copy(
                src_ref=y_ref.at[:, pl.ds(j * N_PER, N_PER)],
                dst_ref=out_ref.at[pl.ds(my * M_PER, M_PER), :],
                send_sem=send_sems.at[h - 1],
                recv_sem=recv_sems.at[my],
                device_id=(j,),
                device_id_type=pl.DeviceIdType.MESH,
            )
            rdma.start()
            sends.append(rdma)

        for h in range(1, N_DEV):
            p = lax.rem(my + N_DEV - h, N_DEV)
            recv = pltpu.make_async_remote_copy(
                src_ref=y_ref.at[:, pl.ds(p * N_PER, N_PER)],
                dst_ref=out_ref.at[pl.ds(p * M_PER, M_PER), :],
                send_sem=send_sems.at[h - 1],
                recv_sem=recv_sems.at[p],
                device_id=(p,),
                device_id_type=pl.DeviceIdType.MESH,
            )
            recv.wait_recv()

        for rdma in sends:
            rdma.wait_send()

    out_shape = jax.ShapeDtypeStruct((N_DEV * M_PER, N_PER), jnp.float32)
    return pl.pallas_call(
        body,
        out_shape=out_shape,
        in_specs=[
            pl.BlockSpec(memory_space=pltpu.VMEM),
            pl.BlockSpec(memory_space=pltpu.MemorySpace.HBM),
            pl.BlockSpec(memory_space=pltpu.SMEM),
            pl.BlockSpec(memory_space=pltpu.SMEM),
        ],
        out_specs=pl.BlockSpec(memory_space=pltpu.VMEM),
        scratch_shapes=[
            pltpu.VMEM((2, K, CHUNK), jnp.float32),
            pltpu.VMEM((M_PER, N_COLS), jnp.float32),
            pltpu.SemaphoreType.DMA((2,)),
            pltpu.SemaphoreType.DMA((N_DEV - 1,)),
            pltpu.SemaphoreType.DMA((N_DEV,)),
        ],
    )(x, w_mat, scale_x, scale_w)
